# baseline (speedup 1.0000x reference)
import functools

import jax
import jax.numpy as jnp
from jax import lax
from jax.experimental import pallas as pl
from jax.experimental.pallas import tpu as pltpu
from jax.experimental.pallas import tpu_sc as plsc

_VOCAB = 64
_HID = 8

_NC = 2
_NS = 16
_NW = _NC * _NS

_C = 512
_L = 16


def _ctab_body(w_ref, tab_ref):
    w_t = jnp.transpose(w_ref[:, :])
    w0 = w_t[0:1, :]
    tb = jnp.round(w0 * 10.0).astype(jnp.int32)
    tb = jnp.maximum(tb, 0)
    idx = tb % _VOCAB
    vals = idx.astype(jnp.float32) / 10.0
    tab_ref[:, :] = jnp.zeros((16, 128), jnp.float32)
    tab_ref[0:1, 0:_VOCAB] = idx.astype(jnp.float32)
    tab_ref[1:2, 0:_VOCAB] = vals
    tab_ref[3:3 + _HID, 0:_VOCAB] = w_t


def _build_ctab(w):
    return pl.pallas_call(
        _ctab_body,
        out_shape=jax.ShapeDtypeStruct((16, 128), jnp.float32),
    )(w)


def _sc_body(n_chunks, seq, ctab_hbm, ids_hbm, lg_hbm, hid_hbm,
             ctab_v, ids_v, blk0, blk1, hblk0, hblk1, sl0, sl1, sh0, sh1):
    b = lax.axis_index("s") * _NC + lax.axis_index("c")

    pltpu.sync_copy(ctab_hbm, ctab_v)
    pltpu.sync_copy(ids_hbm.at[b, :], ids_v)

    def zrow(row, carry):
        z = jnp.zeros((_L,), jnp.float32)
        for col in range(0, _C, _L):
            blk0[row, pl.ds(col, _L)] = z
            blk1[row, pl.ds(col, _L)] = z
        return carry
    lax.fori_loop(0, _VOCAB, zrow, 0)

    lane = lax.iota(jnp.int32, _L)
    bufs = ((blk0, hblk0, sl0, sh0), (blk1, hblk1, sl1, sh1))

    def scatter(blk, base, val_row):
        for k in range(_C // _L):
            ids16 = ids_v[pl.ds(base + k * _L, _L)]
            r0 = jnp.zeros((_L,), jnp.int32)
            idx16 = plsc.load_gather(ctab_v, [r0, ids16]).astype(jnp.int32)
            val16 = plsc.load_gather(ctab_v, [r0 + val_row, ids16])
            col16 = lane + (k * _L)
            plsc.store_scatter(blk, [idx16, col16], val16)

    def hfill(hblk, base):
        for k in range(_C // _L):
            ids16 = ids_v[pl.ds(base + k * _L, _L)]
            r0 = jnp.zeros((_L,), jnp.int32)
            for h in range(_HID):
                vh = plsc.load_gather(ctab_v, [r0 + (3 + h), ids16])
                hblk[h, pl.ds(k * _L, _L)] = vh

    def step(g, carry):
        for buf, (blk, hblk, sl, sh) in enumerate(bufs):
            i = g * 2 + buf
            s0 = i * _C

            @pl.when(g > 0)
            def _recycle():
                pltpu.make_async_copy(blk, lg_hbm.at[b, :, pl.ds(s0, _C)],
                                      sl).wait()
                pltpu.make_async_copy(hblk, hid_hbm.at[b, :, pl.ds(s0, _C)],
                                      sh).wait()
                scatter(blk, (i - 2) * _C, 2)

            scatter(blk, s0, 1)
            pltpu.async_copy(blk, lg_hbm.at[b, :, pl.ds(s0, _C)], sl)
            hfill(hblk, s0)
            pltpu.async_copy(hblk, hid_hbm.at[b, :, pl.ds(s0, _C)], sh)
        return carry

    lax.fori_loop(0, n_chunks // 2, step, 0)

    for blk, hblk, sl, sh in bufs:
        pltpu.make_async_copy(blk, lg_hbm.at[b, :, pl.ds(0, _C)], sl).wait()
        pltpu.make_async_copy(hblk, hid_hbm.at[b, :, pl.ds(0, _C)], sh).wait()


def _sc_run(ctab, ids2d, bsz, seq):
    assert bsz == _NW and seq % (2 * _C) == 0
    n_chunks = seq // _C
    mesh = plsc.VectorSubcoreMesh(core_axis_name="c", subcore_axis_name="s")
    k = functools.partial(
        pl.kernel,
        out_type=[
            jax.ShapeDtypeStruct((bsz, _VOCAB, seq), jnp.float32),
            jax.ShapeDtypeStruct((bsz, _HID, seq), jnp.float32),
        ],
        mesh=mesh,
        compiler_params=pltpu.CompilerParams(needs_layout_passes=False),
        scratch_types=[
            pltpu.VMEM((16, 128), jnp.float32),
            pltpu.VMEM((seq,), jnp.int32),
            pltpu.VMEM((_VOCAB, _C), jnp.float32),
            pltpu.VMEM((_VOCAB, _C), jnp.float32),
            pltpu.VMEM((_HID, _C), jnp.float32),
            pltpu.VMEM((_HID, _C), jnp.float32),
            pltpu.SemaphoreType.DMA,
            pltpu.SemaphoreType.DMA,
            pltpu.SemaphoreType.DMA,
            pltpu.SemaphoreType.DMA,
        ],
    )(functools.partial(_sc_body, n_chunks, seq))
    return k(ctab, ids2d)


def kernel(input_ids, embedding_weight):
    bsz, seq = input_ids.shape
    ids2d = input_ids.astype(jnp.int32)
    ctab = _build_ctab(embedding_weight)
    lg_t, hid_t = _sc_run(ctab, ids2d, bsz, seq)
    logits = jnp.swapaxes(lg_t, 1, 2)
    hidden = jnp.swapaxes(hid_t, 1, 2)
    return logits, hidden


# speedup vs baseline: 1.5255x; 1.5255x over previous
import functools

import jax
import jax.numpy as jnp
from jax import lax
from jax.experimental import pallas as pl
from jax.experimental.pallas import tpu as pltpu
from jax.experimental.pallas import tpu_sc as plsc

_VOCAB = 64
_HID = 8

_NC = 2
_NS = 16
_NW = _NC * _NS

_C = 512
_L = 16


def _ctab_body(w_ref, tab_ref):
    w_t = jnp.transpose(w_ref[:, :])
    w0 = w_t[0:1, :]
    tb = jnp.round(w0 * 10.0).astype(jnp.int32)
    tb = jnp.maximum(tb, 0)
    idx = tb % _VOCAB
    vals = idx.astype(jnp.float32) / 10.0
    tab_ref[:, :] = jnp.zeros((16, 128), jnp.float32)
    tab_ref[0:1, 0:_VOCAB] = idx.astype(jnp.float32)
    tab_ref[1:2, 0:_VOCAB] = vals
    tab_ref[3:3 + _HID, 0:_VOCAB] = w_t


def _build_ctab(w):
    return pl.pallas_call(
        _ctab_body,
        out_shape=jax.ShapeDtypeStruct((16, 128), jnp.float32),
    )(w)


def _sc_body(n_chunks, seq, ctab_hbm, ids_hbm, lg_hbm, hid_hbm,
             ctab_v, ids_v, blk0, blk1, hblk0, hblk1, sl0, sl1, sh0, sh1):
    b = lax.axis_index("s") * _NC + lax.axis_index("c")

    pltpu.sync_copy(ctab_hbm, ctab_v)
    pltpu.sync_copy(ids_hbm.at[b, :], ids_v)

    def zrow(row, carry):
        z = jnp.zeros((_L,), jnp.float32)
        for col in range(0, _C, _L):
            blk0[row, pl.ds(col, _L)] = z
            blk1[row, pl.ds(col, _L)] = z
        return carry
    lax.fori_loop(0, _VOCAB, zrow, 0)

    lane = lax.iota(jnp.int32, _L)
    bufs = ((blk0, hblk0, sl0, sh0), (blk1, hblk1, sl1, sh1))

    def scatter(blk, base, zero):
        @plsc.parallel_loop(0, _C, _L, unroll=4)
        def _(off):
            ids16 = ids_v[pl.ds(base + off, _L)]
            r0 = jnp.zeros((_L,), jnp.int32)
            idx_f = plsc.load_gather(ctab_v, [r0, ids16])
            idx16 = idx_f.astype(jnp.int32)
            if zero:
                val16 = jnp.zeros((_L,), jnp.float32)
            else:
                val16 = idx_f / 10.0
            plsc.store_scatter(blk, [idx16, lane + off], val16)

    def hfill(hblk, base):
        @plsc.parallel_loop(0, _C, _L, unroll=2)
        def _(off):
            ids16 = ids_v[pl.ds(base + off, _L)]
            r0 = jnp.zeros((_L,), jnp.int32)
            for h in range(_HID):
                vh = plsc.load_gather(ctab_v, [r0 + (3 + h), ids16])
                hblk[h, pl.ds(off, _L)] = vh

    def step(g, carry):
        for buf, (blk, hblk, sl, sh) in enumerate(bufs):
            i = g * 2 + buf
            s0 = i * _C

            @pl.when(g > 0)
            def _recycle():
                pltpu.make_async_copy(blk, lg_hbm.at[b, :, pl.ds(s0, _C)],
                                      sl).wait()
                pltpu.make_async_copy(hblk, hid_hbm.at[b, :, pl.ds(s0, _C)],
                                      sh).wait()
                scatter(blk, (i - 2) * _C, zero=True)

            scatter(blk, s0, zero=False)
            pltpu.async_copy(blk, lg_hbm.at[b, :, pl.ds(s0, _C)], sl)
            hfill(hblk, s0)
            pltpu.async_copy(hblk, hid_hbm.at[b, :, pl.ds(s0, _C)], sh)
        return carry

    lax.fori_loop(0, n_chunks // 2, step, 0)

    for blk, hblk, sl, sh in bufs:
        pltpu.make_async_copy(blk, lg_hbm.at[b, :, pl.ds(0, _C)], sl).wait()
        pltpu.make_async_copy(hblk, hid_hbm.at[b, :, pl.ds(0, _C)], sh).wait()


def _sc_run(ctab, ids2d, bsz, seq):
    assert bsz == _NW and seq % (2 * _C) == 0
    n_chunks = seq // _C
    mesh = plsc.VectorSubcoreMesh(core_axis_name="c", subcore_axis_name="s")
    k = functools.partial(
        pl.kernel,
        out_type=[
            jax.ShapeDtypeStruct((bsz, _VOCAB, seq), jnp.float32),
            jax.ShapeDtypeStruct((bsz, _HID, seq), jnp.float32),
        ],
        mesh=mesh,
        compiler_params=pltpu.CompilerParams(needs_layout_passes=False),
        scratch_types=[
            pltpu.VMEM((16, 128), jnp.float32),
            pltpu.VMEM((seq,), jnp.int32),
            pltpu.VMEM((_VOCAB, _C), jnp.float32),
            pltpu.VMEM((_VOCAB, _C), jnp.float32),
            pltpu.VMEM((_HID, _C), jnp.float32),
            pltpu.VMEM((_HID, _C), jnp.float32),
            pltpu.SemaphoreType.DMA,
            pltpu.SemaphoreType.DMA,
            pltpu.SemaphoreType.DMA,
            pltpu.SemaphoreType.DMA,
        ],
    )(functools.partial(_sc_body, n_chunks, seq))
    return k(ctab, ids2d)


def kernel(input_ids, embedding_weight):
    bsz, seq = input_ids.shape
    ids2d = input_ids.astype(jnp.int32)
    ctab = _build_ctab(embedding_weight)
    lg_t, hid_t = _sc_run(ctab, ids2d, bsz, seq)
    logits = jnp.swapaxes(lg_t, 1, 2)
    hidden = jnp.swapaxes(hid_t, 1, 2)
    return logits, hidden
